# baseline (speedup 1.0000x reference)
import jax
import jax.numpy as jnp
from jax import lax
from jax.experimental import pallas as pl
from jax.experimental.pallas import tpu as pltpu
from jax.experimental.pallas import tpu_sc as plsc

_B = 4096 * 200
_D = 64
_NW = 32
_BPW = _B // _NW
_CHUNK = 1024
_NCHUNK = _BPW // _CHUNK


def _emb_body(x_hbm, table_hbm, out_hbm, idx_v, rows_v, sem):
    wid = lax.axis_index("s") * 2 + lax.axis_index("c")
    base = wid * _BPW

    def round_body(g, carry):
        off = base + g * _CHUNK
        pltpu.sync_copy(x_hbm.at[pl.ds(off, _CHUNK)], idx_v)
        pltpu.async_copy(table_hbm.at[idx_v], rows_v, sem).wait()
        pltpu.sync_copy(rows_v, out_hbm.at[pl.ds(off, _CHUNK)])
        return carry

    lax.fori_loop(0, _NCHUNK, round_body, 0)


@jax.jit
def kernel(x, table):
    xf = x.reshape(_B)
    mesh = plsc.VectorSubcoreMesh(core_axis_name="c", subcore_axis_name="s")
    out = pl.kernel(
        _emb_body,
        mesh=mesh,
        compiler_params=pltpu.CompilerParams(use_tc_tiling_on_sc=False),
        out_type=jax.ShapeDtypeStruct((_B, _D), jnp.float32),
        scratch_types=[
            pltpu.VMEM((_CHUNK,), jnp.int32),
            pltpu.VMEM((_CHUNK, _D), jnp.float32),
            pltpu.SemaphoreType.DMA,
        ],
    )(xf, table)
    return out.reshape(x.shape + (table.shape[1],))


# speedup vs baseline: 1.0134x; 1.0134x over previous
import jax
import jax.numpy as jnp
from jax import lax
from jax.experimental import pallas as pl
from jax.experimental.pallas import tpu as pltpu
from jax.experimental.pallas import tpu_sc as plsc

_B = 4096 * 200
_D = 64
_NW = 32
_BPW = _B // _NW
_CHUNK = 800
_NROUND = _BPW // _CHUNK
_NPAIR = _NROUND // 2


def _emb_body(x_hbm, table_hbm, out_hbm, idx_v, rows_v,
              s_i0, s_i1, s_g0, s_g1, s_o0, s_o1):
    s_idx = (s_i0, s_i1)
    s_gat = (s_g0, s_g1)
    s_out = (s_o0, s_o1)
    wid = lax.axis_index("s") * 2 + lax.axis_index("c")
    base = wid * _BPW

    def idx_cp(g, b):
        return pltpu.make_async_copy(
            x_hbm.at[pl.ds(base + g * _CHUNK, _CHUNK)], idx_v.at[b], s_idx[b])

    def gat_cp(b):
        return pltpu.make_async_copy(
            table_hbm.at[idx_v.at[b]], rows_v.at[b], s_gat[b])

    def out_cp(g, b):
        return pltpu.make_async_copy(
            rows_v.at[b], out_hbm.at[pl.ds(base + g * _CHUNK, _CHUNK)], s_out[b])

    idx_cp(0, 0).start()
    idx_cp(1, 1).start()

    def pair(p, carry):
        for b in range(2):
            g = p * 2 + b

            @pl.when(p >= 1)
            def _():
                out_cp(g, b).wait()

            idx_cp(g, b).wait()
            gat_cp(b).start()

            if b == 0:
                @pl.when(p >= 1)
                def _():
                    gat_cp(1).wait()
                    out_cp(g - 1, 1).start()
                    idx_cp(g + 1, 1).start()
            else:
                gat_cp(0).wait()
                out_cp(g - 1, 0).start()

                @pl.when(p <= _NPAIR - 2)
                def _():
                    idx_cp(g + 1, 0).start()
        return carry

    lax.fori_loop(0, _NPAIR, pair, 0)

    last = _NROUND - 1
    gat_cp(1).wait()
    out_cp(last, 1).start()
    out_cp(last - 1, 0).wait()
    out_cp(last, 1).wait()


@jax.jit
def kernel(x, table):
    xf = x.reshape(_B)
    mesh = plsc.VectorSubcoreMesh(core_axis_name="c", subcore_axis_name="s")
    out = pl.kernel(
        _emb_body,
        mesh=mesh,
        compiler_params=pltpu.CompilerParams(use_tc_tiling_on_sc=False),
        out_type=jax.ShapeDtypeStruct((_B, _D), jnp.float32),
        scratch_types=[
            pltpu.VMEM((2, _CHUNK), jnp.int32),
            pltpu.VMEM((2, _CHUNK, _D), jnp.float32),
            pltpu.SemaphoreType.DMA,
            pltpu.SemaphoreType.DMA,
            pltpu.SemaphoreType.DMA,
            pltpu.SemaphoreType.DMA,
            pltpu.SemaphoreType.DMA,
            pltpu.SemaphoreType.DMA,
        ],
    )(xf, table)
    return out.reshape(x.shape + (table.shape[1],))
